# baseline (speedup 1.0000x reference)
import functools

import jax
import jax.numpy as jnp
from jax import lax
from jax.experimental import pallas as pl
from jax.experimental.pallas import tpu as pltpu
from jax.experimental.pallas import tpu_sc as plsc

_N = 50000
_D = 512
_C = 8
_NCHUNKS = _N // _C
_NC = 2
_NS = 16
_NW = _NC * _NS
_MAXCH = (_NCHUNKS + _NW - 1) // _NW
_LANES = 16
_RING = 3


def _sc_body(lb_hbm, ub_hbm, nid_hbm, out_hbm, idx_v, lbb, ubb, gsems, osems):
    wid = lax.axis_index("s") * _NC + lax.axis_index("c")
    chunk_lo = (wid * _NCHUNKS) // _NW
    chunk_hi = ((wid + 1) * _NCHUNKS) // _NW
    nchunks = chunk_hi - chunk_lo
    row_lo = chunk_lo * _C

    pltpu.sync_copy(nid_hbm.at[pl.ds(row_lo, _MAXCH * _C)], idx_v)

    def gather(k, slot):
        idxs = idx_v.at[pl.ds(k * _C, _C)]
        pltpu.async_copy(lb_hbm.at[idxs], lbb.at[slot], gsems.at[slot, 0])
        pltpu.async_copy(ub_hbm.at[idxs], ubb.at[slot], gsems.at[slot, 1])

    def gather_wait(slot):
        pltpu.make_async_copy(lb_hbm.at[idx_v.at[pl.ds(0, _C)]],
                              lbb.at[slot], gsems.at[slot, 0]).wait()
        pltpu.make_async_copy(ub_hbm.at[idx_v.at[pl.ds(0, _C)]],
                              ubb.at[slot], gsems.at[slot, 1]).wait()

    def store(k, slot):
        row = row_lo + k * _C
        pltpu.async_copy(lbb.at[slot], out_hbm.at[0, pl.ds(row, _C)],
                         osems.at[slot, 0])
        pltpu.async_copy(ubb.at[slot], out_hbm.at[1, pl.ds(row, _C)],
                         osems.at[slot, 1])

    def store_wait(slot):
        pltpu.make_async_copy(lbb.at[slot], out_hbm.at[0, pl.ds(0, _C)],
                              osems.at[slot, 0]).wait()
        pltpu.make_async_copy(ubb.at[slot], out_hbm.at[1, pl.ds(0, _C)],
                              osems.at[slot, 1]).wait()

    def compute(slot):
        def row_body(r, _):
            for j in range(_D // _LANES):
                sl = pl.ds(j * _LANES, _LANES)
                lv = lbb[slot, r, sl]
                uv = ubb[slot, r, sl]
                ubb[slot, r, sl] = jnp.where(lv == 0.0, 0.0,
                                             jnp.maximum(uv, 0.0))
                lbb[slot, r, sl] = jnp.where((lv + uv) > 0.0, lv, 0.0)
            return _
        lax.fori_loop(0, _C, row_body, 0, unroll=False)

    for p in range(_RING - 1):
        @pl.when(p < nchunks)
        def _(p=p):
            gather(p, p)

    def step(k, _):
        slot = lax.rem(k, _RING)

        @pl.when(k + _RING - 1 < nchunks)
        def _prefetch():
            pslot = lax.rem(k + _RING - 1, _RING)

            @pl.when(k >= 1)
            def _():
                store_wait(pslot)
            gather(k + _RING - 1, pslot)

        gather_wait(slot)
        compute(slot)
        store(k, slot)
        return _

    lax.fori_loop(0, nchunks, step, 0, unroll=False)

    def drain(c, _):
        store_wait(lax.rem(c, _RING))
        return _

    lax.fori_loop(jnp.maximum(nchunks - _RING, 0), nchunks, drain, 0,
                  unroll=False)


@jax.jit
def _sc_call(lb, ub, node_id):
    mesh = plsc.VectorSubcoreMesh(core_axis_name="c", subcore_axis_name="s")
    f = functools.partial(
        pl.kernel,
        out_type=jax.ShapeDtypeStruct((2, _N, _D), jnp.float32),
        mesh=mesh,
        scratch_types=[
            pltpu.VMEM((_MAXCH * _C,), jnp.int32),
            pltpu.VMEM((_RING, _C, _D), jnp.float32),
            pltpu.VMEM((_RING, _C, _D), jnp.float32),
            pltpu.SemaphoreType.DMA((_RING, 2)),
            pltpu.SemaphoreType.DMA((_RING, 2)),
        ],
    )(_sc_body)
    return f(lb, ub, node_id)


def kernel(lb, ub, node_id):
    return _sc_call(lb, ub, node_id)


# speedup vs baseline: 2.5579x; 2.5579x over previous
import functools

import jax
import jax.numpy as jnp
from jax import lax
from jax.experimental import pallas as pl
from jax.experimental.pallas import tpu as pltpu
from jax.experimental.pallas import tpu_sc as plsc

_N = 50000
_D = 512
_C = 16
_NCHUNKS = _N // _C
_NC = 2
_NS = 16
_NW = _NC * _NS
_MAXCH = (_NCHUNKS + _NW - 1) // _NW
_LANES = 16
_RING = 6


def _sc_body(lb_hbm, ub_hbm, nid_hbm, out_hbm, idx_v, lbb, ubb, gsems, osems):
    wid = lax.axis_index("s") * _NC + lax.axis_index("c")
    chunk_lo = (wid * _NCHUNKS) // _NW
    chunk_hi = ((wid + 1) * _NCHUNKS) // _NW
    nchunks = chunk_hi - chunk_lo
    row_lo = chunk_lo * _C

    pltpu.sync_copy(nid_hbm.at[pl.ds(row_lo, _MAXCH * _C)], idx_v)

    def gather(k, slot):
        idxs = idx_v.at[pl.ds(k * _C, _C)]
        pltpu.async_copy(lb_hbm.at[idxs], lbb.at[slot], gsems.at[slot, 0])
        pltpu.async_copy(ub_hbm.at[idxs], ubb.at[slot], gsems.at[slot, 1])

    def gather_wait(slot):
        pltpu.make_async_copy(lb_hbm.at[idx_v.at[pl.ds(0, _C)]],
                              lbb.at[slot], gsems.at[slot, 0]).wait()
        pltpu.make_async_copy(ub_hbm.at[idx_v.at[pl.ds(0, _C)]],
                              ubb.at[slot], gsems.at[slot, 1]).wait()

    def store(k, slot):
        row = row_lo + k * _C
        pltpu.async_copy(lbb.at[slot], out_hbm.at[0, pl.ds(row, _C)],
                         osems.at[slot, 0])
        pltpu.async_copy(ubb.at[slot], out_hbm.at[1, pl.ds(row, _C)],
                         osems.at[slot, 1])

    def store_wait(slot):
        pltpu.make_async_copy(lbb.at[slot], out_hbm.at[0, pl.ds(0, _C)],
                              osems.at[slot, 0]).wait()
        pltpu.make_async_copy(ubb.at[slot], out_hbm.at[1, pl.ds(0, _C)],
                              osems.at[slot, 1]).wait()

    def compute(slot):
        def row_body(r, _):
            for j in range(_D // _LANES):
                sl = pl.ds(j * _LANES, _LANES)
                lv = lbb[slot, r, sl]
                uv = ubb[slot, r, sl]
                ubb[slot, r, sl] = jnp.where(lv == 0.0, 0.0,
                                             jnp.maximum(uv, 0.0))
                lbb[slot, r, sl] = jnp.where((lv + uv) > 0.0, lv, 0.0)
            return _
        lax.fori_loop(0, _C, row_body, 0, unroll=False)

    for p in range(_RING - 1):
        @pl.when(p < nchunks)
        def _(p=p):
            gather(p, p)

    def step(k, _):
        slot = lax.rem(k, _RING)

        @pl.when(k + _RING - 1 < nchunks)
        def _prefetch():
            pslot = lax.rem(k + _RING - 1, _RING)

            @pl.when(k >= 1)
            def _():
                store_wait(pslot)
            gather(k + _RING - 1, pslot)

        gather_wait(slot)
        compute(slot)
        store(k, slot)
        return _

    lax.fori_loop(0, nchunks, step, 0, unroll=False)

    def drain(c, _):
        store_wait(lax.rem(c, _RING))
        return _

    lax.fori_loop(jnp.maximum(nchunks - _RING, 0), nchunks, drain, 0,
                  unroll=False)


@jax.jit
def _sc_call(lb, ub, node_id):
    mesh = plsc.VectorSubcoreMesh(core_axis_name="c", subcore_axis_name="s")
    f = functools.partial(
        pl.kernel,
        out_type=jax.ShapeDtypeStruct((2, _N, _D), jnp.float32),
        mesh=mesh,
        scratch_types=[
            pltpu.VMEM((_MAXCH * _C,), jnp.int32),
            pltpu.VMEM((_RING, _C, _D), jnp.float32),
            pltpu.VMEM((_RING, _C, _D), jnp.float32),
            pltpu.SemaphoreType.DMA((_RING, 2)),
            pltpu.SemaphoreType.DMA((_RING, 2)),
        ],
    )(_sc_body)
    return f(lb, ub, node_id)


def kernel(lb, ub, node_id):
    return _sc_call(lb, ub, node_id)


# speedup vs baseline: 2.5988x; 1.0160x over previous
import functools

import jax
import jax.numpy as jnp
from jax import lax
from jax.experimental import pallas as pl
from jax.experimental.pallas import tpu as pltpu
from jax.experimental.pallas import tpu_sc as plsc

_N = 50000
_D = 512
_C = 16
_NCHUNKS = _N // _C
_NC = 2
_NS = 16
_NW = _NC * _NS
_MAXCH = (_NCHUNKS + _NW - 1) // _NW
_LANES = 16
_RING = 3


def _sc_body(lb_hbm, ub_hbm, nid_hbm, out_hbm, idx_v, lbb, ubb, gsems, osems):
    wid = lax.axis_index("s") * _NC + lax.axis_index("c")
    chunk_lo = (wid * _NCHUNKS) // _NW
    chunk_hi = ((wid + 1) * _NCHUNKS) // _NW
    nchunks = chunk_hi - chunk_lo
    row_lo = chunk_lo * _C

    pltpu.sync_copy(nid_hbm.at[pl.ds(row_lo, _MAXCH * _C)], idx_v)

    def gather(k, slot):
        idxs = idx_v.at[pl.ds(k * _C, _C)]
        pltpu.async_copy(lb_hbm.at[idxs], lbb.at[slot], gsems.at[slot, 0])
        pltpu.async_copy(ub_hbm.at[idxs], ubb.at[slot], gsems.at[slot, 1])

    def gather_wait(slot):
        pltpu.make_async_copy(lb_hbm.at[idx_v.at[pl.ds(0, _C)]],
                              lbb.at[slot], gsems.at[slot, 0]).wait()
        pltpu.make_async_copy(ub_hbm.at[idx_v.at[pl.ds(0, _C)]],
                              ubb.at[slot], gsems.at[slot, 1]).wait()

    def store(k, slot):
        row = row_lo + k * _C
        pltpu.async_copy(lbb.at[slot], out_hbm.at[0, pl.ds(row, _C)],
                         osems.at[slot, 0])
        pltpu.async_copy(ubb.at[slot], out_hbm.at[1, pl.ds(row, _C)],
                         osems.at[slot, 1])

    def store_wait(slot):
        pltpu.make_async_copy(lbb.at[slot], out_hbm.at[0, pl.ds(0, _C)],
                              osems.at[slot, 0]).wait()
        pltpu.make_async_copy(ubb.at[slot], out_hbm.at[1, pl.ds(0, _C)],
                              osems.at[slot, 1]).wait()

    def compute(slot):
        def row_body(r, _):
            for j in range(_D // _LANES):
                sl = pl.ds(j * _LANES, _LANES)
                lv = lbb[slot, r, sl]
                uv = ubb[slot, r, sl]
                ubb[slot, r, sl] = jnp.where(lv == 0.0, 0.0,
                                             jnp.maximum(uv, 0.0))
                lbb[slot, r, sl] = jnp.where((lv + uv) > 0.0, lv, 0.0)
            return _
        lax.fori_loop(0, _C, row_body, 0, unroll=False)

    for p in range(_RING - 1):
        @pl.when(p < nchunks)
        def _(p=p):
            gather(p, p)

    def step(k, _):
        slot = lax.rem(k, _RING)

        @pl.when(k + _RING - 1 < nchunks)
        def _prefetch():
            pslot = lax.rem(k + _RING - 1, _RING)

            @pl.when(k >= 1)
            def _():
                store_wait(pslot)
            gather(k + _RING - 1, pslot)

        gather_wait(slot)
        compute(slot)
        store(k, slot)
        return _

    lax.fori_loop(0, nchunks, step, 0, unroll=False)

    def drain(c, _):
        store_wait(lax.rem(c, _RING))
        return _

    lax.fori_loop(jnp.maximum(nchunks - _RING, 0), nchunks, drain, 0,
                  unroll=False)


@jax.jit
def _sc_call(lb, ub, node_id):
    mesh = plsc.VectorSubcoreMesh(core_axis_name="c", subcore_axis_name="s")
    f = functools.partial(
        pl.kernel,
        out_type=jax.ShapeDtypeStruct((2, _N, _D), jnp.float32),
        mesh=mesh,
        scratch_types=[
            pltpu.VMEM((_MAXCH * _C,), jnp.int32),
            pltpu.VMEM((_RING, _C, _D), jnp.float32),
            pltpu.VMEM((_RING, _C, _D), jnp.float32),
            pltpu.SemaphoreType.DMA((_RING, 2)),
            pltpu.SemaphoreType.DMA((_RING, 2)),
        ],
    )(_sc_body)
    return f(lb, ub, node_id)


def kernel(lb, ub, node_id):
    return _sc_call(lb, ub, node_id)
